# baseline (speedup 1.0000x reference)
import functools

import jax
import jax.numpy as jnp
from jax import lax
from jax.experimental import pallas as pl
from jax.experimental.pallas import tpu as pltpu
from jax.experimental.pallas import tpu_sc as plsc

D = 768
P, Q, S = 7, 1, 69
N = 5
T = P + Q + S
NC, NS = 2, 16
NW = NC * NS

S_CHUNKS = ((0, 24), (24, 24), (48, 21))
MAX_ROWS = 24

_JOBS = []
for _i in range(N):
    for _off, _sz in S_CHUNKS:
        _JOBS.append(("s", _i, _off, _sz, P + Q + _off))
for _i in range(N):
    _JOBS.append(("p", _i, 0, P, 0))
for _i in range(N):
    _JOBS.append(("q", _i, 0, Q, P))

_mesh = plsc.VectorSubcoreMesh(core_axis_name="c", subcore_axis_name="s")


@functools.partial(
    pl.kernel,
    mesh=_mesh,
    out_type=jax.ShapeDtypeStruct((N, T, D), jnp.float32),
    scratch_types=[
        pltpu.VMEM((24, D), jnp.float32),
        pltpu.VMEM((21, D), jnp.float32),
        pltpu.VMEM((P, D), jnp.float32),
        pltpu.VMEM((1, D), jnp.float32),
    ],
)
def _sc_concat(p_hbm, q_hbm, s_hbm, o_hbm, b24, b21, b7, b1):
    wid = lax.axis_index("s") * NC + lax.axis_index("c")
    for jid, (kind, i, off, rows, dst) in enumerate(_JOBS):

        @pl.when(wid == (jid % NW))
        def _(kind=kind, i=i, off=off, rows=rows, dst=dst):
            if kind == "q":
                pltpu.sync_copy(q_hbm.at[pl.ds(i, 1), :], b1)
                pltpu.sync_copy(b1, o_hbm.at[i, pl.ds(dst, 1), :])
            else:
                buf = {24: b24, 21: b21, P: b7}[rows]
                src = (p_hbm if kind == "p" else s_hbm).at[i, pl.ds(off, rows), :]
                pltpu.sync_copy(src, buf)
                pltpu.sync_copy(buf, o_hbm.at[i, pl.ds(dst, rows), :])


def kernel(embedding_prefix, learnable_quality, embedding_suffix):
    return _sc_concat(embedding_prefix, learnable_quality, embedding_suffix)


# speedup vs baseline: 5.6163x; 5.6163x over previous
import jax
import jax.numpy as jnp
from jax.experimental import pallas as pl
from jax.experimental.pallas import tpu as pltpu

D = 768
P, Q, S = 7, 1, 69
N = 5
T = P + Q + S
CHUNKS = ((0, 16), (16, 16), (32, 16), (48, 21))


def _concat_body(p_ref, q_ref, s_ref, o_ref, v_ref, sem_in, sem_out):
    ip = pltpu.make_async_copy(p_ref, v_ref.at[:, :P, :], sem_in.at[0])
    iq = pltpu.make_async_copy(q_ref, v_ref.at[:, P, :], sem_in.at[1])
    ip.start()
    iq.start()

    i_s = []
    for k, (off, sz) in enumerate(CHUNKS):
        i_s.append(
            pltpu.make_async_copy(
                s_ref.at[:, off : off + sz, :],
                v_ref.at[:, P + Q + off : P + Q + off + sz, :],
                sem_in.at[2 + k],
            )
        )
    i_s[0].start()

    ip.wait()
    iq.wait()
    outs = []
    for k, (off, sz) in enumerate(CHUNKS):
        if k + 1 < len(CHUNKS):
            i_s[k + 1].start()
        i_s[k].wait()
        if k == 0:
            lo, hi = 0, P + Q + sz
        else:
            lo, hi = P + Q + off, P + Q + off + sz
        ok = pltpu.make_async_copy(
            v_ref.at[:, lo:hi, :], o_ref.at[:, lo:hi, :], sem_out.at[k]
        )
        ok.start()
        outs.append(ok)
    for c in outs:
        c.wait()


def kernel(embedding_prefix, learnable_quality, embedding_suffix):
    return pl.pallas_call(
        _concat_body,
        out_shape=jax.ShapeDtypeStruct((N, T, D), jnp.float32),
        in_specs=[
            pl.BlockSpec(memory_space=pl.ANY),
            pl.BlockSpec(memory_space=pl.ANY),
            pl.BlockSpec(memory_space=pl.ANY),
        ],
        out_specs=pl.BlockSpec(memory_space=pl.ANY),
        scratch_shapes=[
            pltpu.VMEM((N, T, D), jnp.float32),
            pltpu.SemaphoreType.DMA((2 + len(CHUNKS),)),
            pltpu.SemaphoreType.DMA((len(CHUNKS),)),
        ],
    )(embedding_prefix, learnable_quality, embedding_suffix)


# speedup vs baseline: 9.7461x; 1.7353x over previous
import jax
import jax.numpy as jnp
from jax.experimental import pallas as pl
from jax.experimental.pallas import tpu as pltpu

D = 768
P, Q, S = 7, 1, 69
N = 5
T = P + Q + S
CHUNKS = ((0, 24), (24, 24), (48, 21))


def _concat_body(p_ref, q_ref, s_ref, o_ref, v_ref, sem_in, sem_out):
    ip = pltpu.make_async_copy(p_ref, v_ref.at[:, :P, :], sem_in.at[0])
    iq = pltpu.make_async_copy(q_ref, v_ref.at[:, P, :], sem_in.at[1])
    i_s = [
        pltpu.make_async_copy(
            s_ref.at[:, off : off + sz, :],
            v_ref.at[:, P + Q + off : P + Q + off + sz, :],
            sem_in.at[2 + k],
        )
        for k, (off, sz) in enumerate(CHUNKS)
    ]
    ip.start()
    iq.start()
    for c in i_s:
        c.start()

    ip.wait()
    iq.wait()
    outs = []
    for k, (off, sz) in enumerate(CHUNKS):
        i_s[k].wait()
        lo = 0 if k == 0 else P + Q + off
        hi = P + Q + off + sz
        ok = pltpu.make_async_copy(
            v_ref.at[:, lo:hi, :], o_ref.at[:, lo:hi, :], sem_out.at[k]
        )
        ok.start()
        outs.append(ok)
    for c in outs:
        c.wait()


def kernel(embedding_prefix, learnable_quality, embedding_suffix):
    return pl.pallas_call(
        _concat_body,
        out_shape=jax.ShapeDtypeStruct((N, T, D), jnp.float32),
        in_specs=[
            pl.BlockSpec(memory_space=pl.ANY),
            pl.BlockSpec(memory_space=pl.ANY),
            pl.BlockSpec(memory_space=pl.ANY),
        ],
        out_specs=pl.BlockSpec(memory_space=pl.ANY),
        scratch_shapes=[
            pltpu.VMEM((N, T, D), jnp.float32),
            pltpu.SemaphoreType.DMA((2 + len(CHUNKS),)),
            pltpu.SemaphoreType.DMA((len(CHUNKS),)),
        ],
    )(embedding_prefix, learnable_quality, embedding_suffix)
